# baseline (speedup 1.0000x reference)
import functools

import jax
import jax.numpy as jnp
import numpy as np
from jax import lax
from jax.experimental import pallas as pl
from jax.experimental.pallas import tpu as pltpu
from jax.experimental.pallas import tpu_sc as plsc

_NC, _NS = 2, 16
_NW = _NC * _NS
_BT = 256
_E = 8


def _softplus(v):
    return jnp.maximum(v, 0.0) + jnp.log1p(jnp.exp(-jnp.abs(v)))


def _ssm_body(x_ref, wd_ref, bd_ref, wb_ref, wc_ref, alt_ref, dp_ref, out_ref,
              out16_ref, h_ref, barA_ref, bbx_ref, hall_ref, *, ch):
    i = pl.program_id(0)

    @pl.when(i == 0)
    def _():
        h_ref[...] = jnp.zeros_like(h_ref)

    xc = x_ref[...]
    delta = _softplus(
        lax.dot_general(xc, wd_ref[...], (((1,), (1,)), ((), ())),
                        preferred_element_type=jnp.float32) + bd_ref[...])
    Bc = lax.dot_general(xc, wb_ref[...], (((1,), (1,)), ((), ())),
                         preferred_element_type=jnp.float32)
    Cc = lax.dot_general(xc, wc_ref[...], (((1,), (1,)), ((), ())),
                         preferred_element_type=jnp.float32)
    At = -jnp.exp(alt_ref[...])

    barA_ref[...] = jnp.exp(delta[:, None, :] * At[None, :, :])
    bbx_ref[...] = (jnp.clip(delta[:, None, :] * Bc[:, :, None], -10.0, 10.0)
                    * xc[:, None, :])

    def step(t, h):
        h = barA_ref[t] * h + bbx_ref[t]
        h = jnp.clip(h, -10000.0, 10000.0)
        hall_ref[t] = h
        return h

    h_ref[...] = lax.fori_loop(0, ch, step, h_ref[...])

    y = jnp.sum(hall_ref[...] * Cc[:, :, None], axis=1)
    o = y + xc * dp_ref[...]
    out_ref[...] = o
    out16_ref[...] = o.astype(jnp.bfloat16)


def _run_ssm(xf, Wd, bd, WB, WC, A_logT, Dp):
    L, D = xf.shape
    N = WB.shape[0]
    CH = 128
    grid = (L // CH,)
    return pl.pallas_call(
        functools.partial(_ssm_body, ch=CH),
        grid=grid,
        in_specs=[
            pl.BlockSpec((CH, D), lambda i: (i, 0)),
            pl.BlockSpec((D, D), lambda i: (0, 0)),
            pl.BlockSpec((1, D), lambda i: (0, 0)),
            pl.BlockSpec((N, D), lambda i: (0, 0)),
            pl.BlockSpec((N, D), lambda i: (0, 0)),
            pl.BlockSpec((N, D), lambda i: (0, 0)),
            pl.BlockSpec((1, D), lambda i: (0, 0)),
        ],
        out_specs=[pl.BlockSpec((CH, D), lambda i: (i, 0)),
                   pl.BlockSpec((CH, D), lambda i: (i, 0))],
        out_shape=[jax.ShapeDtypeStruct((L, D), jnp.float32),
                   jax.ShapeDtypeStruct((L, D), jnp.bfloat16)],
        scratch_shapes=[
            pltpu.VMEM((N, D), jnp.float32),
            pltpu.VMEM((CH, N, D), jnp.float32),
            pltpu.VMEM((CH, N, D), jnp.float32),
            pltpu.VMEM((CH, N, D), jnp.float32),
        ],
        compiler_params=pltpu.CompilerParams(
            dimension_semantics=("arbitrary",)),
    )(xf, Wd, bd.reshape(1, D), WB, WC, A_logT, Dp.reshape(1, D))


def _route_body(flat_ref, wr_ref, tri_ref, pos1_ref, pos2_ref, w12_ref,
                bexp_ref, *, nblk, bt):
    T = flat_ref.shape[0]
    E = wr_ref.shape[0]
    logits = lax.dot_general(flat_ref[...], wr_ref[...],
                             (((1,), (1,)), ((), ())),
                             preferred_element_type=jnp.float32)
    m = jnp.max(logits, axis=-1, keepdims=True)
    p = jnp.exp(logits - m)
    p = p / jnp.sum(p, axis=-1, keepdims=True)
    eidx = lax.broadcasted_iota(jnp.int32, p.shape, 1)
    m1 = jnp.max(p, axis=-1, keepdims=True)
    i1 = jnp.min(jnp.where(p == m1, eidx, E), axis=-1, keepdims=True)
    oh1 = (eidx == i1).astype(jnp.float32)
    pm = jnp.where(oh1 > 0, -jnp.inf, p)
    m2 = jnp.max(pm, axis=-1, keepdims=True)
    i2 = jnp.min(jnp.where(pm == m2, eidx, E), axis=-1, keepdims=True)
    oh2 = (eidx == i2).astype(jnp.float32)
    denom = m1 + m2 + 1e-9
    w12_ref[...] = jnp.concatenate([m1 / denom, m2 / denom], axis=1)

    oh12 = jnp.concatenate([oh1, oh2], axis=1).astype(jnp.bfloat16)
    c12 = lax.dot_general(tri_ref[...], oh12, (((1,), (0,)), ((), ())),
                          preferred_element_type=jnp.float32)
    c1 = c12[:, :E]
    c2 = c12[:, E:]
    cnt0 = jnp.sum(oh1, axis=0, keepdims=True)
    cnt = cnt0 + jnp.sum(oh2, axis=0, keepdims=True)
    nblk_e = jnp.floor((cnt + (bt - 1)) * (1.0 / bt))
    s_io = lax.broadcasted_iota(jnp.int32, (E, E), 0)
    t_io = lax.broadcasted_iota(jnp.int32, (E, E), 1)
    ident = (s_io == t_io).astype(jnp.float32)
    lower = (t_io < s_io).astype(jnp.float32)
    nblk_c = lax.dot_general(ident, nblk_e,
                             (((1,), (1,)), ((), ())),
                             preferred_element_type=jnp.float32)
    blkstart_c = lax.dot_general(lower, nblk_c,
                                 (((1,), (0,)), ((), ())),
                                 preferred_element_type=jnp.float32)
    rowstart_c = blkstart_c * bt

    base1 = lax.dot_general(oh1, rowstart_c,
                            (((1,), (0,)), ((), ())),
                            preferred_element_type=jnp.float32)
    base2 = lax.dot_general(oh2, rowstart_c,
                            (((1,), (0,)), ((), ())),
                            preferred_element_type=jnp.float32)
    rank1 = jnp.sum(oh1 * (c1 - oh1), axis=1, keepdims=True)
    rank2 = jnp.sum(oh2 * (cnt0 + c2 - oh2), axis=1, keepdims=True)
    pos1_ref[...] = (base1 + rank1).astype(jnp.int32)
    pos2_ref[...] = (base2 + rank2).astype(jnp.int32)

    b_io = lax.broadcasted_iota(jnp.int32, (E, nblk), 1)
    le = (blkstart_c <= b_io.astype(jnp.float32)).astype(jnp.float32)
    bexp_ref[...] = (jnp.sum(le, axis=0, keepdims=True) - 1.0).astype(jnp.int32)


def _run_route(flat, Wr, nblk, bt):
    T, D = flat.shape
    E = Wr.shape[0]
    tri = jnp.asarray(np.tril(np.ones((T, T), np.float32)), jnp.bfloat16)
    return pl.pallas_call(
        functools.partial(_route_body, nblk=nblk, bt=bt),
        in_specs=[pl.BlockSpec((T, D), lambda: (0, 0)),
                  pl.BlockSpec((E, D), lambda: (0, 0)),
                  pl.BlockSpec((T, T), lambda: (0, 0))],
        out_specs=[pl.BlockSpec((T, 1), lambda: (0, 0)),
                   pl.BlockSpec((T, 1), lambda: (0, 0)),
                   pl.BlockSpec((T, 2), lambda: (0, 0)),
                   pl.BlockSpec((1, nblk), lambda: (0, 0))],
        out_shape=[jax.ShapeDtypeStruct((T, 1), jnp.int32),
                   jax.ShapeDtypeStruct((T, 1), jnp.int32),
                   jax.ShapeDtypeStruct((T, 2), jnp.float32),
                   jax.ShapeDtypeStruct((1, nblk), jnp.int32)],
    )(flat, Wr, tri)


def _sc_dispatch(flat, pos1, pos2, nrows):
    T, D = flat.shape
    bpw = T // _NW
    mesh = plsc.VectorSubcoreMesh(core_axis_name="c", subcore_axis_name="s")

    @functools.partial(
        pl.kernel,
        out_type=jax.ShapeDtypeStruct((nrows, D), jnp.int32),
        mesh=mesh,
        scratch_types=[
            pltpu.VMEM((bpw,), jnp.int32),
            pltpu.VMEM((bpw,), jnp.int32),
            pltpu.VMEM((bpw, D), jnp.int32),
            pltpu.SemaphoreType.DMA,
            pltpu.SemaphoreType.DMA,
        ],
    )
    def k(flat_hbm, p1_hbm, p2_hbm, xs_hbm, i1_v, i2_v, rows_v, sem1, sem2):
        wid = lax.axis_index("s") * _NC + lax.axis_index("c")
        base = wid * bpw
        pltpu.sync_copy(p1_hbm.at[pl.ds(base, bpw)], i1_v)
        pltpu.sync_copy(p2_hbm.at[pl.ds(base, bpw)], i2_v)
        pltpu.sync_copy(flat_hbm.at[pl.ds(base, bpw)], rows_v)
        c1 = pltpu.async_copy(rows_v, xs_hbm.at[i1_v], sem1)
        c2 = pltpu.async_copy(rows_v, xs_hbm.at[i2_v], sem2)
        c1.wait()
        c2.wait()

    return k(flat, pos1, pos2)


def _expert_body(be_ref, xs_ref, wg_ref, wu_ref, wdn_ref, eout_ref, *, h):
    xb16 = xs_ref[...]
    g = lax.dot_general(xb16, wg_ref[0], (((1,), (1,)), ((), ())),
                        preferred_element_type=jnp.float32)
    u = lax.dot_general(xb16, wu_ref[0], (((1,), (1,)), ((), ())),
                        preferred_element_type=jnp.float32)
    act = (g * u) / (1.0 + jnp.exp(-g))
    ss = jnp.sum(act * act, axis=-1, keepdims=True)
    inv_rms = lax.rsqrt(ss * (1.0 / h) + 1e-6)
    eo = lax.dot_general(act.astype(jnp.bfloat16), wdn_ref[0],
                         (((1,), (1,)), ((), ())),
                         preferred_element_type=jnp.float32)
    eout_ref[...] = (eo * inv_rms).astype(jnp.bfloat16)


def _run_experts(xs, bexp, Wg, Wu, Wdn, wn_h, nblk):
    nrows, D = xs.shape
    E, H, _ = Wg.shape
    wdn_eff = (Wdn * wn_h[:, None, :]).astype(jnp.bfloat16)
    grid_spec = pltpu.PrefetchScalarGridSpec(
        num_scalar_prefetch=1,
        grid=(nblk,),
        in_specs=[
            pl.BlockSpec((_BT, D), lambda b, be: (b, 0)),
            pl.BlockSpec((1, H, D), lambda b, be: (be[b], 0, 0)),
            pl.BlockSpec((1, H, D), lambda b, be: (be[b], 0, 0)),
            pl.BlockSpec((1, D, H), lambda b, be: (be[b], 0, 0)),
        ],
        out_specs=pl.BlockSpec((_BT, D), lambda b, be: (b, 0)),
    )
    return pl.pallas_call(
        functools.partial(_expert_body, h=float(H)),
        grid_spec=grid_spec,
        out_shape=jax.ShapeDtypeStruct((nrows, D), jnp.bfloat16),
        compiler_params=pltpu.CompilerParams(
            dimension_semantics=("arbitrary",)),
    )(bexp, xs, Wg.astype(jnp.bfloat16), Wu.astype(jnp.bfloat16),
      wdn_eff)


def _sc_combine(eout, pos1, pos2):
    T = pos1.shape[0]
    D = eout.shape[1]
    bpw = T // _NW
    mesh = plsc.VectorSubcoreMesh(core_axis_name="c", subcore_axis_name="s")

    @functools.partial(
        pl.kernel,
        out_type=(jax.ShapeDtypeStruct((T, D), jnp.int32),
                  jax.ShapeDtypeStruct((T, D), jnp.int32)),
        mesh=mesh,
        scratch_types=[
            pltpu.VMEM((bpw,), jnp.int32),
            pltpu.VMEM((bpw,), jnp.int32),
            pltpu.VMEM((bpw, D), jnp.int32),
            pltpu.VMEM((bpw, D), jnp.int32),
            pltpu.SemaphoreType.DMA,
            pltpu.SemaphoreType.DMA,
        ],
    )
    def k(eout_hbm, p1_hbm, p2_hbm, g1_hbm, g2_hbm,
          i1_v, i2_v, r1_v, r2_v, sem1, sem2):
        wid = lax.axis_index("s") * _NC + lax.axis_index("c")
        base = wid * bpw
        pltpu.sync_copy(p1_hbm.at[pl.ds(base, bpw)], i1_v)
        pltpu.sync_copy(p2_hbm.at[pl.ds(base, bpw)], i2_v)
        c1 = pltpu.async_copy(eout_hbm.at[i1_v], r1_v, sem1)
        c2 = pltpu.async_copy(eout_hbm.at[i2_v], r2_v, sem2)
        c1.wait()
        c2.wait()
        pltpu.sync_copy(r1_v, g1_hbm.at[pl.ds(base, bpw)])
        pltpu.sync_copy(r2_v, g2_hbm.at[pl.ds(base, bpw)])

    return k(eout, pos1, pos2)


def _final_body(flat_ref, g1_ref, g2_ref, w12_ref, wn_ref, out_ref):
    w12 = w12_ref[...]
    s = (flat_ref[...] + w12[:, 0:1] * g1_ref[...].astype(jnp.float32)
         + w12[:, 1:2] * g2_ref[...].astype(jnp.float32))
    rms = jnp.sqrt(jnp.mean(s * s, axis=-1, keepdims=True) + 1e-6)
    out_ref[...] = wn_ref[...] * s / rms


def _run_final(flat, g1, g2, w12, wn):
    T, D = flat.shape
    return pl.pallas_call(
        _final_body,
        in_specs=[pl.BlockSpec((T, D), lambda: (0, 0)),
                  pl.BlockSpec((T, D), lambda: (0, 0)),
                  pl.BlockSpec((T, D), lambda: (0, 0)),
                  pl.BlockSpec((T, 2), lambda: (0, 0)),
                  pl.BlockSpec((1, D), lambda: (0, 0))],
        out_specs=pl.BlockSpec((T, D), lambda: (0, 0)),
        out_shape=jax.ShapeDtypeStruct((T, D), jnp.float32),
    )(flat, g1, g2, w12, wn.reshape(1, D))


def kernel(x, A_log, Dp, Wd, bd, WB, WC, Wr, Wg, Wu, Wdn, wn_h, wn):
    B, L, D = x.shape
    E = Wr.shape[0]
    nblk = (2 * L) // _BT + E - 1
    nrows = nblk * _BT
    xf = x.reshape(L, D)
    flat, flat16 = _run_ssm(xf, Wd, bd, WB, WC, A_log.T, Dp)
    pos1, pos2, w12, bexp = _run_route(flat, Wr, nblk, _BT)
    p1 = pos1.reshape(L)
    p2 = pos2.reshape(L)
    flat_i = lax.bitcast_convert_type(flat16.reshape(L, D // 2, 2), jnp.int32)
    xs_i = _sc_dispatch(flat_i, p1, p2, nrows)
    xs16 = lax.bitcast_convert_type(xs_i, jnp.bfloat16).reshape(nrows, D)
    eout = _run_experts(xs16, bexp.reshape(nblk), Wg, Wu, Wdn, wn_h, nblk)
    eout_i = lax.bitcast_convert_type(eout.reshape(nrows, D // 2, 2), jnp.int32)
    g1_i, g2_i = _sc_combine(eout_i, p1, p2)
    g1 = lax.bitcast_convert_type(g1_i, jnp.bfloat16).reshape(L, D)
    g2 = lax.bitcast_convert_type(g2_i, jnp.bfloat16).reshape(L, D)
    out = _run_final(flat, g1, g2, w12, wn)
    return out.reshape(B, L, D)


# speedup vs baseline: 2.3251x; 2.3251x over previous
import functools

import jax
import jax.numpy as jnp
import numpy as np
from jax import lax
from jax.experimental import pallas as pl
from jax.experimental.pallas import tpu as pltpu
from jax.experimental.pallas import tpu_sc as plsc

_NC, _NS = 2, 16
_NW = _NC * _NS
_BT = 256
_E = 8


def _softplus(v):
    return jnp.maximum(v, 0.0) + jnp.log1p(jnp.exp(-jnp.abs(v)))


def _ssm_body(x_ref, wd_ref, bd_ref, wb_ref, wc_ref, alt_ref, dp_ref, out_ref,
              h_ref, barA_ref, bbx_ref, hall_ref, *, ch):
    i = pl.program_id(0)

    @pl.when(i == 0)
    def _():
        h_ref[...] = jnp.zeros_like(h_ref)

    xc = x_ref[...]
    delta = _softplus(
        lax.dot_general(xc, wd_ref[...], (((1,), (1,)), ((), ())),
                        preferred_element_type=jnp.float32) + bd_ref[...])
    Bc = lax.dot_general(xc, wb_ref[...], (((1,), (1,)), ((), ())),
                         preferred_element_type=jnp.float32)
    Cc = lax.dot_general(xc, wc_ref[...], (((1,), (1,)), ((), ())),
                         preferred_element_type=jnp.float32)
    At = -jnp.exp(alt_ref[...])

    barA_ref[...] = jnp.exp(delta[:, None, :] * At[None, :, :])
    bbx_ref[...] = (jnp.clip(delta[:, None, :] * Bc[:, :, None], -10.0, 10.0)
                    * xc[:, None, :])

    def step(t, h):
        h = barA_ref[t] * h + bbx_ref[t]
        h = jnp.clip(h, -10000.0, 10000.0)
        hall_ref[t] = h
        return h

    h_ref[...] = lax.fori_loop(0, ch, step, h_ref[...])

    y = jnp.sum(hall_ref[...] * Cc[:, :, None], axis=1)
    out_ref[...] = y + xc * dp_ref[...]


def _run_ssm(xf, Wd, bd, WB, WC, A_logT, Dp):
    L, D = xf.shape
    N = WB.shape[0]
    CH = 128
    grid = (L // CH,)
    return pl.pallas_call(
        functools.partial(_ssm_body, ch=CH),
        grid=grid,
        in_specs=[
            pl.BlockSpec((CH, D), lambda i: (i, 0)),
            pl.BlockSpec((D, D), lambda i: (0, 0)),
            pl.BlockSpec((1, D), lambda i: (0, 0)),
            pl.BlockSpec((N, D), lambda i: (0, 0)),
            pl.BlockSpec((N, D), lambda i: (0, 0)),
            pl.BlockSpec((N, D), lambda i: (0, 0)),
            pl.BlockSpec((1, D), lambda i: (0, 0)),
        ],
        out_specs=pl.BlockSpec((CH, D), lambda i: (i, 0)),
        out_shape=jax.ShapeDtypeStruct((L, D), jnp.float32),
        scratch_shapes=[
            pltpu.VMEM((N, D), jnp.float32),
            pltpu.VMEM((CH, N, D), jnp.float32),
            pltpu.VMEM((CH, N, D), jnp.float32),
            pltpu.VMEM((CH, N, D), jnp.float32),
        ],
        compiler_params=pltpu.CompilerParams(
            dimension_semantics=("arbitrary",)),
    )(xf, Wd, bd.reshape(1, D), WB, WC, A_logT, Dp.reshape(1, D))


def _route_body(flat_ref, wr_ref, tri_ref, pos1_ref, pos2_ref, w12_ref,
                bexp_ref, *, nblk, bt):
    T = flat_ref.shape[0]
    E = wr_ref.shape[0]
    logits = lax.dot_general(flat_ref[...], wr_ref[...],
                             (((1,), (1,)), ((), ())),
                             preferred_element_type=jnp.float32)
    m = jnp.max(logits, axis=-1, keepdims=True)
    p = jnp.exp(logits - m)
    p = p / jnp.sum(p, axis=-1, keepdims=True)
    eidx = lax.broadcasted_iota(jnp.int32, p.shape, 1)
    m1 = jnp.max(p, axis=-1, keepdims=True)
    i1 = jnp.min(jnp.where(p == m1, eidx, E), axis=-1, keepdims=True)
    oh1 = (eidx == i1).astype(jnp.float32)
    pm = jnp.where(oh1 > 0, -jnp.inf, p)
    m2 = jnp.max(pm, axis=-1, keepdims=True)
    i2 = jnp.min(jnp.where(pm == m2, eidx, E), axis=-1, keepdims=True)
    oh2 = (eidx == i2).astype(jnp.float32)
    denom = m1 + m2 + 1e-9
    w12_ref[...] = jnp.concatenate([m1 / denom, m2 / denom], axis=1)

    oh12 = jnp.concatenate([oh1, oh2], axis=1).astype(jnp.bfloat16)
    c12 = lax.dot_general(tri_ref[...], oh12, (((1,), (0,)), ((), ())),
                          preferred_element_type=jnp.float32)
    c1 = c12[:, :E]
    c2 = c12[:, E:]
    cnt0 = jnp.sum(oh1, axis=0, keepdims=True)
    cnt = cnt0 + jnp.sum(oh2, axis=0, keepdims=True)
    nblk_e = jnp.floor((cnt + (bt - 1)) * (1.0 / bt))
    s_io = lax.broadcasted_iota(jnp.int32, (E, E), 0)
    t_io = lax.broadcasted_iota(jnp.int32, (E, E), 1)
    ident = (s_io == t_io).astype(jnp.float32)
    lower = (t_io < s_io).astype(jnp.float32)
    nblk_c = lax.dot_general(ident, nblk_e,
                             (((1,), (1,)), ((), ())),
                             preferred_element_type=jnp.float32)
    blkstart_c = lax.dot_general(lower, nblk_c,
                                 (((1,), (0,)), ((), ())),
                                 preferred_element_type=jnp.float32)
    rowstart_c = blkstart_c * bt

    base1 = lax.dot_general(oh1, rowstart_c,
                            (((1,), (0,)), ((), ())),
                            preferred_element_type=jnp.float32)
    base2 = lax.dot_general(oh2, rowstart_c,
                            (((1,), (0,)), ((), ())),
                            preferred_element_type=jnp.float32)
    rank1 = jnp.sum(oh1 * (c1 - oh1), axis=1, keepdims=True)
    rank2 = jnp.sum(oh2 * (cnt0 + c2 - oh2), axis=1, keepdims=True)
    pos1_ref[...] = (base1 + rank1).astype(jnp.int32)
    pos2_ref[...] = (base2 + rank2).astype(jnp.int32)

    b_io = lax.broadcasted_iota(jnp.int32, (E, nblk), 1)
    le = (blkstart_c <= b_io.astype(jnp.float32)).astype(jnp.float32)
    bexp_ref[...] = (jnp.sum(le, axis=0, keepdims=True) - 1.0).astype(jnp.int32)


def _run_route(flat, Wr, nblk, bt):
    T, D = flat.shape
    E = Wr.shape[0]
    tri = jnp.asarray(np.tril(np.ones((T, T), np.float32)), jnp.bfloat16)
    return pl.pallas_call(
        functools.partial(_route_body, nblk=nblk, bt=bt),
        in_specs=[pl.BlockSpec((T, D), lambda: (0, 0)),
                  pl.BlockSpec((E, D), lambda: (0, 0)),
                  pl.BlockSpec((T, T), lambda: (0, 0))],
        out_specs=[pl.BlockSpec((T, 1), lambda: (0, 0)),
                   pl.BlockSpec((T, 1), lambda: (0, 0)),
                   pl.BlockSpec((T, 2), lambda: (0, 0)),
                   pl.BlockSpec((1, nblk), lambda: (0, 0))],
        out_shape=[jax.ShapeDtypeStruct((T, 1), jnp.int32),
                   jax.ShapeDtypeStruct((T, 1), jnp.int32),
                   jax.ShapeDtypeStruct((T, 2), jnp.float32),
                   jax.ShapeDtypeStruct((1, nblk), jnp.int32)],
    )(flat, Wr, tri)


def _sc_dispatch(flat, pos1, pos2, nrows):
    T, D = flat.shape
    bpw = T // _NW
    mesh = plsc.VectorSubcoreMesh(core_axis_name="c", subcore_axis_name="s")

    @functools.partial(
        pl.kernel,
        out_type=jax.ShapeDtypeStruct((nrows, D), jnp.float32),
        mesh=mesh,
        scratch_types=[
            pltpu.VMEM((bpw,), jnp.int32),
            pltpu.VMEM((bpw,), jnp.int32),
            pltpu.VMEM((bpw, D), jnp.float32),
            pltpu.SemaphoreType.DMA,
            pltpu.SemaphoreType.DMA,
        ],
    )
    def k(flat_hbm, p1_hbm, p2_hbm, xs_hbm, i1_v, i2_v, rows_v, sem1, sem2):
        wid = lax.axis_index("s") * _NC + lax.axis_index("c")
        base = wid * bpw
        pltpu.sync_copy(p1_hbm.at[pl.ds(base, bpw)], i1_v)
        pltpu.sync_copy(p2_hbm.at[pl.ds(base, bpw)], i2_v)
        pltpu.sync_copy(flat_hbm.at[pl.ds(base, bpw)], rows_v)
        c1 = pltpu.async_copy(rows_v, xs_hbm.at[i1_v], sem1)
        c2 = pltpu.async_copy(rows_v, xs_hbm.at[i2_v], sem2)
        c1.wait()
        c2.wait()

    return k(flat, pos1, pos2)


def _expert_body(be_ref, xs_ref, wg_ref, wu_ref, wdn_ref, eout_ref, *, h):
    xb16 = xs_ref[...].astype(jnp.bfloat16)
    g = lax.dot_general(xb16, wg_ref[0], (((1,), (1,)), ((), ())),
                        preferred_element_type=jnp.float32)
    u = lax.dot_general(xb16, wu_ref[0], (((1,), (1,)), ((), ())),
                        preferred_element_type=jnp.float32)
    act = (g * u) / (1.0 + jnp.exp(-g))
    ss = jnp.sum(act * act, axis=-1, keepdims=True)
    inv_rms = lax.rsqrt(ss * (1.0 / h) + 1e-6)
    eo = lax.dot_general(act.astype(jnp.bfloat16), wdn_ref[0],
                         (((1,), (1,)), ((), ())),
                         preferred_element_type=jnp.float32)
    eout_ref[...] = eo * inv_rms


def _run_experts(xs, bexp, Wg, Wu, Wdn, wn_h, nblk):
    nrows, D = xs.shape
    E, H, _ = Wg.shape
    wdn_eff = (Wdn * wn_h[:, None, :]).astype(jnp.bfloat16)
    grid_spec = pltpu.PrefetchScalarGridSpec(
        num_scalar_prefetch=1,
        grid=(nblk,),
        in_specs=[
            pl.BlockSpec((_BT, D), lambda b, be: (b, 0)),
            pl.BlockSpec((1, H, D), lambda b, be: (be[b], 0, 0)),
            pl.BlockSpec((1, H, D), lambda b, be: (be[b], 0, 0)),
            pl.BlockSpec((1, D, H), lambda b, be: (be[b], 0, 0)),
        ],
        out_specs=pl.BlockSpec((_BT, D), lambda b, be: (b, 0)),
    )
    return pl.pallas_call(
        functools.partial(_expert_body, h=float(H)),
        grid_spec=grid_spec,
        out_shape=jax.ShapeDtypeStruct((nrows, D), jnp.float32),
        compiler_params=pltpu.CompilerParams(
            dimension_semantics=("arbitrary",)),
    )(bexp, xs, Wg.astype(jnp.bfloat16), Wu.astype(jnp.bfloat16),
      wdn_eff)


def _sc_combine(eout, pos1, pos2):
    T = pos1.shape[0]
    D = eout.shape[1]
    bpw = T // _NW
    mesh = plsc.VectorSubcoreMesh(core_axis_name="c", subcore_axis_name="s")

    @functools.partial(
        pl.kernel,
        out_type=(jax.ShapeDtypeStruct((T, D), jnp.float32),
                  jax.ShapeDtypeStruct((T, D), jnp.float32)),
        mesh=mesh,
        scratch_types=[
            pltpu.VMEM((bpw,), jnp.int32),
            pltpu.VMEM((bpw,), jnp.int32),
            pltpu.VMEM((bpw, D), jnp.float32),
            pltpu.VMEM((bpw, D), jnp.float32),
            pltpu.SemaphoreType.DMA,
            pltpu.SemaphoreType.DMA,
        ],
    )
    def k(eout_hbm, p1_hbm, p2_hbm, g1_hbm, g2_hbm,
          i1_v, i2_v, r1_v, r2_v, sem1, sem2):
        wid = lax.axis_index("s") * _NC + lax.axis_index("c")
        base = wid * bpw
        pltpu.sync_copy(p1_hbm.at[pl.ds(base, bpw)], i1_v)
        pltpu.sync_copy(p2_hbm.at[pl.ds(base, bpw)], i2_v)
        c1 = pltpu.async_copy(eout_hbm.at[i1_v], r1_v, sem1)
        c2 = pltpu.async_copy(eout_hbm.at[i2_v], r2_v, sem2)
        c1.wait()
        c2.wait()
        pltpu.sync_copy(r1_v, g1_hbm.at[pl.ds(base, bpw)])
        pltpu.sync_copy(r2_v, g2_hbm.at[pl.ds(base, bpw)])

    return k(eout, pos1, pos2)


def _final_body(flat_ref, g1_ref, g2_ref, w12_ref, wn_ref, out_ref):
    w12 = w12_ref[...]
    s = (flat_ref[...] + w12[:, 0:1] * g1_ref[...].astype(jnp.float32)
         + w12[:, 1:2] * g2_ref[...].astype(jnp.float32))
    rms = jnp.sqrt(jnp.mean(s * s, axis=-1, keepdims=True) + 1e-6)
    out_ref[...] = wn_ref[...] * s / rms


def _run_final(flat, g1, g2, w12, wn):
    T, D = flat.shape
    return pl.pallas_call(
        _final_body,
        in_specs=[pl.BlockSpec((T, D), lambda: (0, 0)),
                  pl.BlockSpec((T, D), lambda: (0, 0)),
                  pl.BlockSpec((T, D), lambda: (0, 0)),
                  pl.BlockSpec((T, 2), lambda: (0, 0)),
                  pl.BlockSpec((1, D), lambda: (0, 0))],
        out_specs=pl.BlockSpec((T, D), lambda: (0, 0)),
        out_shape=jax.ShapeDtypeStruct((T, D), jnp.float32),
    )(flat, g1, g2, w12, wn.reshape(1, D))


def kernel(x, A_log, Dp, Wd, bd, WB, WC, Wr, Wg, Wu, Wdn, wn_h, wn):
    B, L, D = x.shape
    E = Wr.shape[0]
    nblk = (2 * L) // _BT + E - 1
    nrows = nblk * _BT
    xf = x.reshape(L, D)
    flat = _run_ssm(xf, Wd, bd, WB, WC, A_log.T, Dp)
    pos1, pos2, w12, bexp = _run_route(flat, Wr, nblk, _BT)
    p1 = pos1.reshape(L)
    p2 = pos2.reshape(L)
    xs = _sc_dispatch(flat, p1, p2, nrows)
    eout = _run_experts(xs, bexp.reshape(nblk), Wg, Wu, Wdn, wn_h, nblk)
    g1, g2 = _sc_combine(eout, p1, p2)
    out = _run_final(flat, g1, g2, w12, wn)
    return out.reshape(B, L, D)


# speedup vs baseline: 2.3885x; 1.0273x over previous
import functools

import jax
import jax.numpy as jnp
import numpy as np
from jax import lax
from jax.experimental import pallas as pl
from jax.experimental.pallas import tpu as pltpu
from jax.experimental.pallas import tpu_sc as plsc

_NC, _NS = 2, 16
_NW = _NC * _NS
_BT = 256
_E = 8


def _softplus(v):
    return jnp.maximum(v, 0.0) + jnp.log1p(jnp.exp(-jnp.abs(v)))


def _ssm_body(x_ref, wd_ref, bd_ref, wb_ref, wc_ref, alt_ref, dp_ref, out_ref,
              h_ref, barA_ref, bbx_ref, hall_ref, *, ch):
    i = pl.program_id(0)

    @pl.when(i == 0)
    def _():
        h_ref[...] = jnp.zeros_like(h_ref)

    xc = x_ref[...]
    delta = _softplus(
        lax.dot_general(xc, wd_ref[...], (((1,), (1,)), ((), ())),
                        preferred_element_type=jnp.float32) + bd_ref[...])
    Bc = lax.dot_general(xc, wb_ref[...], (((1,), (1,)), ((), ())),
                         preferred_element_type=jnp.float32)
    Cc = lax.dot_general(xc, wc_ref[...], (((1,), (1,)), ((), ())),
                         preferred_element_type=jnp.float32)
    At = -jnp.exp(alt_ref[...])

    barA_ref[...] = jnp.exp(delta[:, None, :] * At[None, :, :])
    bbx_ref[...] = (jnp.clip(delta[:, None, :] * Bc[:, :, None], -10.0, 10.0)
                    * xc[:, None, :])

    def step(t, h):
        h = barA_ref[t] * h + bbx_ref[t]
        h = jnp.clip(h, -10000.0, 10000.0)
        hall_ref[t] = h
        return h

    h_ref[...] = lax.fori_loop(0, ch, step, h_ref[...])

    y = jnp.sum(hall_ref[...] * Cc[:, :, None], axis=1)
    out_ref[...] = y + xc * dp_ref[...]


def _run_ssm(xf, Wd, bd, WB, WC, A_logT, Dp):
    L, D = xf.shape
    N = WB.shape[0]
    CH = 256
    grid = (L // CH,)
    return pl.pallas_call(
        functools.partial(_ssm_body, ch=CH),
        grid=grid,
        in_specs=[
            pl.BlockSpec((CH, D), lambda i: (i, 0)),
            pl.BlockSpec((D, D), lambda i: (0, 0)),
            pl.BlockSpec((1, D), lambda i: (0, 0)),
            pl.BlockSpec((N, D), lambda i: (0, 0)),
            pl.BlockSpec((N, D), lambda i: (0, 0)),
            pl.BlockSpec((N, D), lambda i: (0, 0)),
            pl.BlockSpec((1, D), lambda i: (0, 0)),
        ],
        out_specs=pl.BlockSpec((CH, D), lambda i: (i, 0)),
        out_shape=jax.ShapeDtypeStruct((L, D), jnp.float32),
        scratch_shapes=[
            pltpu.VMEM((N, D), jnp.float32),
            pltpu.VMEM((CH, N, D), jnp.float32),
            pltpu.VMEM((CH, N, D), jnp.float32),
            pltpu.VMEM((CH, N, D), jnp.float32),
        ],
        compiler_params=pltpu.CompilerParams(
            dimension_semantics=("arbitrary",)),
    )(xf, Wd, bd.reshape(1, D), WB, WC, A_logT, Dp.reshape(1, D))


def _route_body(flat_ref, wr_ref, tri_ref, pos1_ref, pos2_ref, w12_ref,
                bexp_ref, *, nblk, bt):
    T = flat_ref.shape[0]
    E = wr_ref.shape[0]
    logits = lax.dot_general(flat_ref[...], wr_ref[...],
                             (((1,), (1,)), ((), ())),
                             preferred_element_type=jnp.float32)
    m = jnp.max(logits, axis=-1, keepdims=True)
    p = jnp.exp(logits - m)
    p = p / jnp.sum(p, axis=-1, keepdims=True)
    eidx = lax.broadcasted_iota(jnp.int32, p.shape, 1)
    m1 = jnp.max(p, axis=-1, keepdims=True)
    i1 = jnp.min(jnp.where(p == m1, eidx, E), axis=-1, keepdims=True)
    oh1 = (eidx == i1).astype(jnp.float32)
    pm = jnp.where(oh1 > 0, -jnp.inf, p)
    m2 = jnp.max(pm, axis=-1, keepdims=True)
    i2 = jnp.min(jnp.where(pm == m2, eidx, E), axis=-1, keepdims=True)
    oh2 = (eidx == i2).astype(jnp.float32)
    denom = m1 + m2 + 1e-9
    w12_ref[...] = jnp.concatenate([m1 / denom, m2 / denom], axis=1)

    oh12 = jnp.concatenate([oh1, oh2], axis=1).astype(jnp.bfloat16)
    c12 = lax.dot_general(tri_ref[...], oh12, (((1,), (0,)), ((), ())),
                          preferred_element_type=jnp.float32)
    c1 = c12[:, :E]
    c2 = c12[:, E:]
    cnt0 = jnp.sum(oh1, axis=0, keepdims=True)
    cnt = cnt0 + jnp.sum(oh2, axis=0, keepdims=True)
    nblk_e = jnp.floor((cnt + (bt - 1)) * (1.0 / bt))
    s_io = lax.broadcasted_iota(jnp.int32, (E, E), 0)
    t_io = lax.broadcasted_iota(jnp.int32, (E, E), 1)
    ident = (s_io == t_io).astype(jnp.float32)
    lower = (t_io < s_io).astype(jnp.float32)
    nblk_c = lax.dot_general(ident, nblk_e,
                             (((1,), (1,)), ((), ())),
                             preferred_element_type=jnp.float32)
    blkstart_c = lax.dot_general(lower, nblk_c,
                                 (((1,), (0,)), ((), ())),
                                 preferred_element_type=jnp.float32)
    rowstart_c = blkstart_c * bt

    base1 = lax.dot_general(oh1, rowstart_c,
                            (((1,), (0,)), ((), ())),
                            preferred_element_type=jnp.float32)
    base2 = lax.dot_general(oh2, rowstart_c,
                            (((1,), (0,)), ((), ())),
                            preferred_element_type=jnp.float32)
    rank1 = jnp.sum(oh1 * (c1 - oh1), axis=1, keepdims=True)
    rank2 = jnp.sum(oh2 * (cnt0 + c2 - oh2), axis=1, keepdims=True)
    pos1_ref[...] = (base1 + rank1).astype(jnp.int32)
    pos2_ref[...] = (base2 + rank2).astype(jnp.int32)

    b_io = lax.broadcasted_iota(jnp.int32, (E, nblk), 1)
    le = (blkstart_c <= b_io.astype(jnp.float32)).astype(jnp.float32)
    bexp_ref[...] = (jnp.sum(le, axis=0, keepdims=True) - 1.0).astype(jnp.int32)


def _run_route(flat, Wr, nblk, bt):
    T, D = flat.shape
    E = Wr.shape[0]
    tri = jnp.asarray(np.tril(np.ones((T, T), np.float32)), jnp.bfloat16)
    return pl.pallas_call(
        functools.partial(_route_body, nblk=nblk, bt=bt),
        in_specs=[pl.BlockSpec((T, D), lambda: (0, 0)),
                  pl.BlockSpec((E, D), lambda: (0, 0)),
                  pl.BlockSpec((T, T), lambda: (0, 0))],
        out_specs=[pl.BlockSpec((T, 1), lambda: (0, 0)),
                   pl.BlockSpec((T, 1), lambda: (0, 0)),
                   pl.BlockSpec((T, 2), lambda: (0, 0)),
                   pl.BlockSpec((1, nblk), lambda: (0, 0))],
        out_shape=[jax.ShapeDtypeStruct((T, 1), jnp.int32),
                   jax.ShapeDtypeStruct((T, 1), jnp.int32),
                   jax.ShapeDtypeStruct((T, 2), jnp.float32),
                   jax.ShapeDtypeStruct((1, nblk), jnp.int32)],
    )(flat, Wr, tri)


def _sc_dispatch(flat, pos1, pos2, nrows):
    T, D = flat.shape
    bpw = T // _NW
    mesh = plsc.VectorSubcoreMesh(core_axis_name="c", subcore_axis_name="s")

    @functools.partial(
        pl.kernel,
        out_type=jax.ShapeDtypeStruct((nrows, D), jnp.float32),
        mesh=mesh,
        scratch_types=[
            pltpu.VMEM((bpw,), jnp.int32),
            pltpu.VMEM((bpw,), jnp.int32),
            pltpu.VMEM((bpw, D), jnp.float32),
            pltpu.SemaphoreType.DMA,
            pltpu.SemaphoreType.DMA,
        ],
    )
    def k(flat_hbm, p1_hbm, p2_hbm, xs_hbm, i1_v, i2_v, rows_v, sem1, sem2):
        wid = lax.axis_index("s") * _NC + lax.axis_index("c")
        base = wid * bpw
        pltpu.sync_copy(p1_hbm.at[pl.ds(base, bpw)], i1_v)
        pltpu.sync_copy(p2_hbm.at[pl.ds(base, bpw)], i2_v)
        pltpu.sync_copy(flat_hbm.at[pl.ds(base, bpw)], rows_v)
        c1 = pltpu.async_copy(rows_v, xs_hbm.at[i1_v], sem1)
        c2 = pltpu.async_copy(rows_v, xs_hbm.at[i2_v], sem2)
        c1.wait()
        c2.wait()

    return k(flat, pos1, pos2)


def _expert_body(be_ref, xs_ref, wg_ref, wu_ref, wdn_ref, eout_ref, *, h):
    xb16 = xs_ref[...].astype(jnp.bfloat16)
    g = lax.dot_general(xb16, wg_ref[0], (((1,), (1,)), ((), ())),
                        preferred_element_type=jnp.float32)
    u = lax.dot_general(xb16, wu_ref[0], (((1,), (1,)), ((), ())),
                        preferred_element_type=jnp.float32)
    act = (g * u) / (1.0 + jnp.exp(-g))
    ss = jnp.sum(act * act, axis=-1, keepdims=True)
    inv_rms = lax.rsqrt(ss * (1.0 / h) + 1e-6)
    eo = lax.dot_general(act.astype(jnp.bfloat16), wdn_ref[0],
                         (((1,), (1,)), ((), ())),
                         preferred_element_type=jnp.float32)
    eout_ref[...] = eo * inv_rms


def _run_experts(xs, bexp, Wg, Wu, Wdn, wn_h, nblk):
    nrows, D = xs.shape
    E, H, _ = Wg.shape
    wdn_eff = (Wdn * wn_h[:, None, :]).astype(jnp.bfloat16)
    grid_spec = pltpu.PrefetchScalarGridSpec(
        num_scalar_prefetch=1,
        grid=(nblk,),
        in_specs=[
            pl.BlockSpec((_BT, D), lambda b, be: (b, 0)),
            pl.BlockSpec((1, H, D), lambda b, be: (be[b], 0, 0)),
            pl.BlockSpec((1, H, D), lambda b, be: (be[b], 0, 0)),
            pl.BlockSpec((1, D, H), lambda b, be: (be[b], 0, 0)),
        ],
        out_specs=pl.BlockSpec((_BT, D), lambda b, be: (b, 0)),
    )
    return pl.pallas_call(
        functools.partial(_expert_body, h=float(H)),
        grid_spec=grid_spec,
        out_shape=jax.ShapeDtypeStruct((nrows, D), jnp.float32),
        compiler_params=pltpu.CompilerParams(
            dimension_semantics=("arbitrary",)),
    )(bexp, xs, Wg.astype(jnp.bfloat16), Wu.astype(jnp.bfloat16),
      wdn_eff)


def _sc_combine(eout, pos1, pos2):
    T = pos1.shape[0]
    D = eout.shape[1]
    bpw = T // _NW
    mesh = plsc.VectorSubcoreMesh(core_axis_name="c", subcore_axis_name="s")

    @functools.partial(
        pl.kernel,
        out_type=(jax.ShapeDtypeStruct((T, D), jnp.float32),
                  jax.ShapeDtypeStruct((T, D), jnp.float32)),
        mesh=mesh,
        scratch_types=[
            pltpu.VMEM((bpw,), jnp.int32),
            pltpu.VMEM((bpw,), jnp.int32),
            pltpu.VMEM((bpw, D), jnp.float32),
            pltpu.VMEM((bpw, D), jnp.float32),
            pltpu.SemaphoreType.DMA,
            pltpu.SemaphoreType.DMA,
        ],
    )
    def k(eout_hbm, p1_hbm, p2_hbm, g1_hbm, g2_hbm,
          i1_v, i2_v, r1_v, r2_v, sem1, sem2):
        wid = lax.axis_index("s") * _NC + lax.axis_index("c")
        base = wid * bpw
        pltpu.sync_copy(p1_hbm.at[pl.ds(base, bpw)], i1_v)
        pltpu.sync_copy(p2_hbm.at[pl.ds(base, bpw)], i2_v)
        c1 = pltpu.async_copy(eout_hbm.at[i1_v], r1_v, sem1)
        c2 = pltpu.async_copy(eout_hbm.at[i2_v], r2_v, sem2)
        c1.wait()
        c2.wait()
        pltpu.sync_copy(r1_v, g1_hbm.at[pl.ds(base, bpw)])
        pltpu.sync_copy(r2_v, g2_hbm.at[pl.ds(base, bpw)])

    return k(eout, pos1, pos2)


def _final_body(flat_ref, g1_ref, g2_ref, w12_ref, wn_ref, out_ref):
    w12 = w12_ref[...]
    s = (flat_ref[...] + w12[:, 0:1] * g1_ref[...].astype(jnp.float32)
         + w12[:, 1:2] * g2_ref[...].astype(jnp.float32))
    rms = jnp.sqrt(jnp.mean(s * s, axis=-1, keepdims=True) + 1e-6)
    out_ref[...] = wn_ref[...] * s / rms


def _run_final(flat, g1, g2, w12, wn):
    T, D = flat.shape
    return pl.pallas_call(
        _final_body,
        in_specs=[pl.BlockSpec((T, D), lambda: (0, 0)),
                  pl.BlockSpec((T, D), lambda: (0, 0)),
                  pl.BlockSpec((T, D), lambda: (0, 0)),
                  pl.BlockSpec((T, 2), lambda: (0, 0)),
                  pl.BlockSpec((1, D), lambda: (0, 0))],
        out_specs=pl.BlockSpec((T, D), lambda: (0, 0)),
        out_shape=jax.ShapeDtypeStruct((T, D), jnp.float32),
    )(flat, g1, g2, w12, wn.reshape(1, D))


def kernel(x, A_log, Dp, Wd, bd, WB, WC, Wr, Wg, Wu, Wdn, wn_h, wn):
    B, L, D = x.shape
    E = Wr.shape[0]
    nblk = (2 * L) // _BT + E - 1
    nrows = nblk * _BT
    xf = x.reshape(L, D)
    flat = _run_ssm(xf, Wd, bd, WB, WC, A_log.T, Dp)
    pos1, pos2, w12, bexp = _run_route(flat, Wr, nblk, _BT)
    p1 = pos1.reshape(L)
    p2 = pos2.reshape(L)
    xs = _sc_dispatch(flat, p1, p2, nrows)
    eout = _run_experts(xs, bexp.reshape(nblk), Wg, Wu, Wdn, wn_h, nblk)
    g1, g2 = _sc_combine(eout, p1, p2)
    out = _run_final(flat, g1, g2, w12, wn)
    return out.reshape(B, L, D)
